# baseline (speedup 1.0000x reference)
import functools

import jax
import jax.numpy as jnp
from jax import lax
from jax.experimental import pallas as pl
from jax.experimental.pallas import tpu as pltpu
from jax.experimental.pallas import tpu_sc as plsc

NUM_CORES = 2
NUM_SUBCORES = 16
NUM_WORKERS = NUM_CORES * NUM_SUBCORES

LANES = 16
CHUNK = 128
GRP = 8
NBUF = 2


def _build_sc_gather(total_rows: int, features: int):
    assert total_rows % (NUM_WORKERS * CHUNK) == 0
    rows_per_w = total_rows // NUM_WORKERS
    chunks_per_w = rows_per_w // CHUNK
    n_chunks = total_rows // CHUNK
    words = CHUNK * features
    shift = (features - 1).bit_length()

    mesh = plsc.VectorSubcoreMesh(
        core_axis_name="c", subcore_axis_name="s",
        num_cores=NUM_CORES, num_subcores=NUM_SUBCORES)

    @functools.partial(
        pl.kernel,
        out_type=jax.ShapeDtypeStruct((n_chunks, words), jnp.float32),
        mesh=mesh,
        scratch_types=[
            pltpu.VMEM((rows_per_w,), jnp.int32),
            pltpu.VMEM((NBUF, words), jnp.int32),
            pltpu.VMEM((NBUF, words), jnp.float32),
            [pltpu.SemaphoreType.DMA] * NBUF,
            [pltpu.SemaphoreType.DMA] * NBUF,
        ],
        compiler_params=pltpu.CompilerParams(use_tc_tiling_on_sc=False,
                                             needs_layout_passes=False),
    )
    def sc_gather(idx_hbm, tab_hbm, out_hbm, idx_v, offs, buf, gsems, wsems):
        wid = lax.axis_index("s") * NUM_CORES + lax.axis_index("c")
        chunk0 = wid * chunks_per_w
        pltpu.sync_copy(idx_hbm.at[wid], idx_v)
        iota = lax.iota(jnp.int32, LANES)

        def fire_chunk(j, b):
            base = j * CHUNK

            @pl.loop(0, CHUNK // GRP)
            def _(g):
                for r in range(GRP):
                    row = g * GRP + r
                    rep = plsc.load_gather(
                        idx_v, [jnp.broadcast_to(base + row, (LANES,))])
                    o0 = lax.shift_left(rep, shift) + iota
                    for h in range(features // LANES):
                        offs[b, pl.ds(row * features + h * LANES, LANES)] = (
                            o0 + h * LANES)

            pltpu.async_copy(tab_hbm.at[offs.at[b]], buf.at[b], gsems[b])

        def drain_chunk(b):
            pltpu.make_async_copy(tab_hbm.at[offs.at[b]], buf.at[b],
                                  gsems[b]).wait()

        def wait_write(b):
            pltpu.make_async_copy(buf.at[b], out_hbm.at[chunk0],
                                  wsems[b]).wait()

        for b in range(NBUF):
            fire_chunk(b, b)

        @pl.loop(0, chunks_per_w // NBUF)
        def _(jo):
            j0 = jo * NBUF
            for b in range(NBUF):
                drain_chunk(b)
                pltpu.async_copy(buf.at[b], out_hbm.at[chunk0 + j0 + b],
                                 wsems[b])
            for b in range(NBUF):
                nj = j0 + NBUF + b

                @pl.when(nj < chunks_per_w)
                def _():
                    wait_write(b)
                    fire_chunk(nj, b)

        for b in range(NBUF):
            wait_write(b)

    return sc_gather


def kernel(inputs, embedding):
    b, s = inputs.shape
    total = b * s
    feats = embedding.shape[1]
    idx2d = inputs.reshape(NUM_WORKERS, -1).astype(jnp.int32)
    gather = _build_sc_gather(total, feats)
    out = gather(idx2d, embedding.reshape(-1))
    return out.reshape(b, s, feats)


# speedup vs baseline: 1.3974x; 1.3974x over previous
import functools

import jax
import jax.numpy as jnp
from jax import lax
from jax.experimental import pallas as pl
from jax.experimental.pallas import tpu as pltpu
from jax.experimental.pallas import tpu_sc as plsc

NUM_CORES = 2
NUM_SUBCORES = 16
NUM_WORKERS = NUM_CORES * NUM_SUBCORES

LANES = 16
CHUNK = 128
GRP = 8
NBUF = 2


def _build_sc_gather(total_rows: int, features: int):
    assert total_rows % (NUM_WORKERS * CHUNK) == 0
    rows_per_w = total_rows // NUM_WORKERS
    chunks_per_w = rows_per_w // CHUNK
    n_chunks = total_rows // CHUNK
    halves = features // LANES
    shift = (features - 1).bit_length()

    mesh = plsc.VectorSubcoreMesh(
        core_axis_name="c", subcore_axis_name="s",
        num_cores=NUM_CORES, num_subcores=NUM_SUBCORES)

    @functools.partial(
        pl.kernel,
        out_type=jax.ShapeDtypeStruct((n_chunks, CHUNK * 2, LANES),
                                      jnp.float32),
        mesh=mesh,
        scratch_types=[
            pltpu.VMEM((rows_per_w,), jnp.int32),
            pltpu.VMEM((NBUF, CHUNK * 2, LANES), jnp.float32),
            [[pltpu.SemaphoreType.DMA] * 4] * NBUF,
            [pltpu.SemaphoreType.DMA] * NBUF,
        ],
        compiler_params=pltpu.CompilerParams(use_tc_tiling_on_sc=False,
                                             needs_layout_passes=False),
    )
    def sc_gather(idx_hbm, tab_hbm, out_hbm, idx_v, buf, gsems, wsems):
        wid = lax.axis_index("s") * NUM_CORES + lax.axis_index("c")
        chunk0 = wid * chunks_per_w
        pltpu.sync_copy(idx_hbm.at[wid], idx_v)
        iota = lax.iota(jnp.int32, LANES)

        def fire_chunk(j, b):
            base = j * CHUNK

            @pl.loop(0, CHUNK // 32)
            def _(g):
                for q in range(4):
                    r0 = g * 32 + q * 8
                    pos = base + r0 + lax.shift_right_logical(iota, 1)
                    idxg = plsc.load_gather(idx_v, [pos])
                    o = lax.shift_left(idxg, 1) + lax.rem(iota, 2)
                    pltpu.async_copy(
                        tab_hbm.at[o], buf.at[b, pl.ds(r0 * 2, 16)],
                        gsems[b][q])

        def drain_chunk(b):
            @pl.loop(0, CHUNK // 32)
            def _(g):
                for q in range(4):
                    pltpu.make_async_copy(
                        tab_hbm.at[iota], buf.at[b, pl.ds(0, 16)],
                        gsems[b][q]).wait()

        def wait_write(b):
            pltpu.make_async_copy(buf.at[b], out_hbm.at[chunk0],
                                  wsems[b]).wait()

        for b in range(NBUF):
            fire_chunk(b, b)

        @pl.loop(0, chunks_per_w // NBUF)
        def _(jo):
            j0 = jo * NBUF
            for b in range(NBUF):
                drain_chunk(b)
                pltpu.async_copy(buf.at[b], out_hbm.at[chunk0 + j0 + b],
                                 wsems[b])
            for b in range(NBUF):
                nj = j0 + NBUF + b

                @pl.when(nj < chunks_per_w)
                def _():
                    wait_write(b)
                    fire_chunk(nj, b)

        for b in range(NBUF):
            wait_write(b)

    return sc_gather


def kernel(inputs, embedding):
    b, s = inputs.shape
    total = b * s
    feats = embedding.shape[1]
    idx2d = inputs.reshape(NUM_WORKERS, -1).astype(jnp.int32)
    gather = _build_sc_gather(total, feats)
    out = gather(idx2d, embedding.reshape(-1, LANES))
    return out.reshape(b, s, feats)


# speedup vs baseline: 1.3994x; 1.0015x over previous
import functools

import jax
import jax.numpy as jnp
from jax import lax
from jax.experimental import pallas as pl
from jax.experimental.pallas import tpu as pltpu
from jax.experimental.pallas import tpu_sc as plsc

NUM_CORES = 2
NUM_SUBCORES = 16
NUM_WORKERS = NUM_CORES * NUM_SUBCORES

LANES = 16
CHUNK = 128
GRP = 8
NBUF = 2


def _build_sc_gather(total_rows: int, features: int):
    assert total_rows % (NUM_WORKERS * CHUNK) == 0
    rows_per_w = total_rows // NUM_WORKERS
    chunks_per_w = rows_per_w // CHUNK
    n_chunks = total_rows // CHUNK
    halves = features // LANES
    shift = (features - 1).bit_length()

    mesh = plsc.VectorSubcoreMesh(
        core_axis_name="c", subcore_axis_name="s",
        num_cores=NUM_CORES, num_subcores=NUM_SUBCORES)

    @functools.partial(
        pl.kernel,
        out_type=jax.ShapeDtypeStruct((n_chunks, CHUNK * 2, LANES),
                                      jnp.float32),
        mesh=mesh,
        scratch_types=[
            pltpu.VMEM((rows_per_w,), jnp.int32),
            pltpu.VMEM((NBUF, CHUNK * 2, LANES), jnp.float32),
            [[pltpu.SemaphoreType.DMA] * 4] * NBUF,
            [pltpu.SemaphoreType.DMA] * NBUF,
        ],
        compiler_params=pltpu.CompilerParams(use_tc_tiling_on_sc=False,
                                             needs_layout_passes=False),
    )
    def sc_gather(idx_hbm, tab_hbm, out_hbm, idx_v, buf, gsems, wsems):
        wid = lax.axis_index("s") * NUM_CORES + lax.axis_index("c")
        chunk0 = wid * chunks_per_w
        pltpu.sync_copy(idx_hbm.at[wid], idx_v)
        iota = lax.iota(jnp.int32, LANES)

        def fire_chunk(j, b):
            base = j * CHUNK

            @pl.loop(0, CHUNK // 32)
            def _(g):
                for q in range(4):
                    r0 = g * 32 + q * 8
                    pos = base + r0 + lax.shift_right_logical(iota, 1)
                    idxg = plsc.load_gather(idx_v, [pos])
                    o = lax.shift_left(idxg, 1) + lax.rem(iota, 2)
                    pltpu.async_copy(
                        tab_hbm.at[plsc.Indices(o, ignored_value=-1)],
                        buf.at[b, pl.ds(r0 * 2, 16)],
                        gsems[b][q])

        def drain_chunk(b):
            @pl.loop(0, CHUNK // 32)
            def _(g):
                for q in range(4):
                    pltpu.make_async_copy(
                        tab_hbm.at[iota], buf.at[b, pl.ds(0, 16)],
                        gsems[b][q]).wait()

        def wait_write(b):
            pltpu.make_async_copy(buf.at[b], out_hbm.at[chunk0],
                                  wsems[b]).wait()

        for b in range(NBUF):
            fire_chunk(b, b)

        @pl.loop(0, chunks_per_w // NBUF)
        def _(jo):
            j0 = jo * NBUF
            for b in range(NBUF):
                drain_chunk(b)
                pltpu.async_copy(buf.at[b], out_hbm.at[chunk0 + j0 + b],
                                 wsems[b])
            for b in range(NBUF):
                nj = j0 + NBUF + b

                @pl.when(nj < chunks_per_w)
                def _():
                    wait_write(b)
                    fire_chunk(nj, b)

        for b in range(NBUF):
            wait_write(b)

    return sc_gather


def kernel(inputs, embedding):
    b, s = inputs.shape
    total = b * s
    feats = embedding.shape[1]
    idx2d = inputs.reshape(NUM_WORKERS, -1).astype(jnp.int32)
    gather = _build_sc_gather(total, feats)
    out = gather(idx2d, embedding.reshape(-1, LANES))
    return out.reshape(b, s, feats)
